# baseline (speedup 1.0000x reference)
import functools

import jax
import jax.numpy as jnp
from jax import lax
from jax.experimental import pallas as pl
from jax.experimental.pallas import tpu as pltpu
import jax.experimental.pallas.tpu_sc as plsc

_B, _N, _D = 4, 8192, 768
_K = 1024
_BLK = 1024
_NBUCK = 2048
_CAP = 16384
_L = 16


def _mono(bits):
    return jnp.where(bits >= 0, bits ^ jnp.int32(-2147483648), ~bits)


def _sc_body(keys_hbm, idx_hbm, ones_hbm,
             keysv, hist, candk, candi, bufk, bufi, outv, onesv):
    cid = lax.axis_index("c")
    sid = lax.axis_index("s")

    del cid
    @pl.when(sid < _B)
    def _():
        row = sid & (_B - 1)
        lane = lax.iota(jnp.int32, _L)
        pltpu.sync_copy(keys_hbm.at[row], keysv)

        for j in range(4 * 256 // _L):
            hist[pl.ds(j * _L, _L)] = jnp.zeros((_L,), jnp.int32)

        @plsc.parallel_loop(0, _N // _L, unroll=4)
        def _h(i):
            v = _mono(plsc.bitcast(keysv[pl.ds((i * _L) & (_N - 1), _L)],
                                   jnp.int32))
            u = plsc.bitcast(v, jnp.uint32)
            d = plsc.bitcast(u >> jnp.uint32(24), jnp.int32)
            cnt, last = plsc.scan_count(d)
            plsc.addupdate_scatter(hist, [d + ((i & 3) << 8)], cnt,
                                   mask=last)

        for j in range(256 // _L):
            hist[pl.ds(j * _L, _L)] = (
                hist[pl.ds(j * _L, _L)]
                + hist[pl.ds(256 + j * _L, _L)]
                + hist[pl.ds(512 + j * _L, _L)]
                + hist[pl.ds(768 + j * _L, _L)])

        def t_body(t, carry):
            total, bfound, nge = carry
            jj = 256 // _L - 1 - t
            v = hist[pl.ds((jj * _L) & 255, _L)]
            rv = lax.rev(v, dimensions=(0,))
            cs0 = plsc.cumsum(rv)
            chunk = jnp.max(cs0)
            cs = cs0 + total
            found = cs >= _K
            b_id = jj * _L + (_L - 1 - lane)
            bc = jnp.max(jnp.where(found, b_id, -1))
            ngc = jnp.min(jnp.where(found, cs, jnp.int32(1 << 30)))
            anyf = jnp.max(jnp.where(found, 1, 0)) > 0
            upd = jnp.logical_and(bfound < 0, anyf)
            bfound = jnp.where(upd, bc, bfound)
            nge = jnp.where(upd, ngc, nge)
            return total + chunk, bfound, nge

        _, bbuck, n = lax.fori_loop(
            0, 256 // _L, t_body,
            (jnp.int32(0), jnp.int32(-1), jnp.int32(0)))

        @plsc.parallel_loop(0, _N // _L, unroll=4, carry=jnp.int32(0))
        def _c(i, wp):
            v = _mono(plsc.bitcast(keysv[pl.ds((i * _L) & (_N - 1), _L)],
                                   jnp.int32))
            u = plsc.bitcast(v, jnp.uint32)
            d = plsc.bitcast(u >> jnp.uint32(24), jnp.int32)
            m = d >= bbuck
            wpm = wp & (_CAP - 1)
            plsc.store_compressed(candk.at[pl.ds(wpm, _L)], v, mask=m)
            plsc.store_compressed(candi.at[pl.ds(wpm, _L)], lane + i * _L,
                                  mask=m)
            return wp + jnp.max(plsc.all_reduce_population_count(m))

        nv = (n + _L - 1) // _L

        def radix_pass(srck, srci, dstk, dsti, shift):
            for j in range(4 * 256 // _L):
                hist[pl.ds(j * _L, _L)] = jnp.zeros((_L,), jnp.int32)

            @plsc.parallel_loop(0, nv, unroll=4)
            def _ph(i):
                v = srck[pl.ds((i * _L) & (_CAP - 1), _L)]
                u = plsc.bitcast(v, jnp.uint32)
                d = plsc.bitcast((u >> jnp.uint32(shift)) & jnp.uint32(255),
                                 jnp.int32)
                valid = (i * _L + lane) < n
                cnt, last = plsc.scan_count(d, mask=valid)
                plsc.addupdate_scatter(hist, [d + ((i & 3) << 8)], cnt,
                                       mask=last)

            carry = jnp.int32(0)
            for j in range(256 // _L):
                v = (hist[pl.ds(j * _L, _L)]
                     + hist[pl.ds(256 + j * _L, _L)]
                     + hist[pl.ds(512 + j * _L, _L)]
                     + hist[pl.ds(768 + j * _L, _L)])
                cs = plsc.cumsum(v)
                hist[pl.ds(j * _L, _L)] = cs - v + carry
                carry = carry + jnp.max(cs)

            def pp(i, c):
                off = (i * _L) & (_CAP - 1)
                vk = srck[pl.ds(off, _L)]
                vi = srci[pl.ds(off, _L)]
                u = plsc.bitcast(vk, jnp.uint32)
                d = plsc.bitcast((u >> jnp.uint32(shift)) & jnp.uint32(255),
                                 jnp.int32)
                valid = (i * _L + lane) < n
                cnt, last = plsc.scan_count(d, mask=valid)
                base = plsc.load_gather(hist, [d])
                addr = base + cnt - 1
                plsc.store_scatter(dstk, [addr], vk, mask=valid)
                plsc.store_scatter(dsti, [addr], vi, mask=valid)
                plsc.store_scatter(hist, [d], base + cnt, mask=last)
                return c

            lax.fori_loop(0, nv, pp, jnp.int32(0))

        radix_pass(candk, candi, bufk, bufi, 0)
        radix_pass(bufk, bufi, candk, candi, 8)
        radix_pass(candk, candi, bufk, bufi, 16)
        radix_pass(bufk, bufi, candk, candi, 24)

        base = n - _K
        for j in range(_K // _L):
            outv[pl.ds(j * _L, _L)] = candi[pl.ds((base + j * _L)
                                                  & (_CAP - 1), _L)]
            onesv[pl.ds(j * _L, _L)] = jnp.full((_L,), 1.0, jnp.float32)
        pltpu.sync_copy(outv, idx_hbm.at[row])
        pltpu.sync_copy(onesv, ones_hbm.at[row])


def _sc_topk(keys):
    mesh = plsc.VectorSubcoreMesh(core_axis_name="c", subcore_axis_name="s",
                                  num_cores=1)
    return pl.kernel(
        _sc_body,
        out_type=(
            jax.ShapeDtypeStruct((_B, _K), jnp.int32),
            jax.ShapeDtypeStruct((_B, _K), jnp.float32),
        ),
        mesh=mesh,
        compiler_params=pltpu.CompilerParams(needs_layout_passes=False),
        scratch_types=[
            pltpu.VMEM((_N,), jnp.float32),
            pltpu.VMEM((_NBUCK,), jnp.int32),
            pltpu.VMEM((_CAP,), jnp.int32),
            pltpu.VMEM((_CAP,), jnp.int32),
            pltpu.VMEM((_CAP,), jnp.int32),
            pltpu.VMEM((_CAP,), jnp.int32),
            pltpu.VMEM((_K,), jnp.int32),
            pltpu.VMEM((_K,), jnp.float32),
        ],
    )(keys)


def kernel(x, routing_token, num_tokens):
    del num_tokens
    scores = jnp.einsum('bnd,d->bn', x, routing_token)
    idx, ones = _sc_topk(scores)
    return ones, idx


# speedup vs baseline: 1.0076x; 1.0076x over previous
import jax
import jax.numpy as jnp
from jax import lax
from jax.experimental import pallas as pl
from jax.experimental.pallas import tpu as pltpu
import jax.experimental.pallas.tpu_sc as plsc

_B, _N, _D = 4, 8192, 768
_K = 1024
_CAP = 16384
_L = 16


def _mono(bits):
    return jnp.where(bits >= 0, bits ^ jnp.int32(-2147483648), ~bits)


def _sc_body(keys_hbm, idx_hbm, ones_hbm,
             keysv, hist, candk, candi, bufk, bufi, outv, onesv):
    cid = lax.axis_index("c")
    sid = lax.axis_index("s")

    del cid
    @pl.when(sid < _B)
    def _():
        row = sid & (_B - 1)
        lane = lax.iota(jnp.int32, _L)
        pltpu.sync_copy(keys_hbm.at[row], keysv)

        for j in range(4 * 256 // _L):
            hist[pl.ds(j * _L, _L)] = jnp.zeros((_L,), jnp.int32)

        @plsc.parallel_loop(0, _N // _L, unroll=4)
        def _h(i):
            v = _mono(plsc.bitcast(keysv[pl.ds((i * _L) & (_N - 1), _L)],
                                   jnp.int32))
            u = plsc.bitcast(v, jnp.uint32)
            d = plsc.bitcast(u >> jnp.uint32(24), jnp.int32)
            cnt, last = plsc.scan_count(d)
            plsc.addupdate_scatter(hist, [d + ((i & 3) << 8)], cnt,
                                   mask=last)

        for j in range(256 // _L):
            hist[pl.ds(j * _L, _L)] = (
                hist[pl.ds(j * _L, _L)]
                + hist[pl.ds(256 + j * _L, _L)]
                + hist[pl.ds(512 + j * _L, _L)]
                + hist[pl.ds(768 + j * _L, _L)])

        def t_body(t, carry):
            total, bfound, nge = carry
            jj = 256 // _L - 1 - t
            v = hist[pl.ds((jj * _L) & 255, _L)]
            rv = lax.rev(v, dimensions=(0,))
            cs0 = plsc.cumsum(rv)
            chunk = jnp.max(cs0)
            cs = cs0 + total
            found = cs >= _K
            b_id = jj * _L + (_L - 1 - lane)
            bc = jnp.max(jnp.where(found, b_id, -1))
            ngc = jnp.min(jnp.where(found, cs, jnp.int32(1 << 30)))
            anyf = jnp.max(jnp.where(found, 1, 0)) > 0
            upd = jnp.logical_and(bfound < 0, anyf)
            bfound = jnp.where(upd, bc, bfound)
            nge = jnp.where(upd, ngc, nge)
            return total + chunk, bfound, nge

        _, bbuck, n = lax.fori_loop(
            0, 256 // _L, t_body,
            (jnp.int32(0), jnp.int32(-1), jnp.int32(0)))

        @plsc.parallel_loop(0, _N // _L, unroll=4, carry=jnp.int32(0))
        def _c(i, wp):
            v = _mono(plsc.bitcast(keysv[pl.ds((i * _L) & (_N - 1), _L)],
                                   jnp.int32))
            u = plsc.bitcast(v, jnp.uint32)
            d = plsc.bitcast(u >> jnp.uint32(24), jnp.int32)
            m = d >= bbuck
            wpm = wp & (_CAP - 1)
            plsc.store_compressed(candk.at[pl.ds(wpm, _L)], v, mask=m)
            plsc.store_compressed(candi.at[pl.ds(wpm, _L)], lane + i * _L,
                                  mask=m)
            return wp + jnp.max(plsc.all_reduce_population_count(m))

        nv = (n + _L - 1) // _L

        def radix_pass(srck, srci, dstk, dsti, shift):
            for j in range(4 * 256 // _L):
                hist[pl.ds(j * _L, _L)] = jnp.zeros((_L,), jnp.int32)

            @plsc.parallel_loop(0, nv, unroll=4)
            def _ph(i):
                v = srck[pl.ds((i * _L) & (_CAP - 1), _L)]
                u = plsc.bitcast(v, jnp.uint32)
                d = plsc.bitcast((u >> jnp.uint32(shift)) & jnp.uint32(255),
                                 jnp.int32)
                valid = (i * _L + lane) < n
                cnt, last = plsc.scan_count(d, mask=valid)
                plsc.addupdate_scatter(hist, [d + ((i & 3) << 8)], cnt,
                                       mask=last)

            carry = jnp.int32(0)
            for j in range(256 // _L):
                v = (hist[pl.ds(j * _L, _L)]
                     + hist[pl.ds(256 + j * _L, _L)]
                     + hist[pl.ds(512 + j * _L, _L)]
                     + hist[pl.ds(768 + j * _L, _L)])
                cs = plsc.cumsum(v)
                hist[pl.ds(j * _L, _L)] = cs - v + carry
                carry = carry + jnp.max(cs)

            def pp(i, c):
                off = (i * _L) & (_CAP - 1)
                vk = srck[pl.ds(off, _L)]
                vi = srci[pl.ds(off, _L)]
                u = plsc.bitcast(vk, jnp.uint32)
                d = plsc.bitcast((u >> jnp.uint32(shift)) & jnp.uint32(255),
                                 jnp.int32)
                valid = (i * _L + lane) < n
                cnt, last = plsc.scan_count(d, mask=valid)
                base = plsc.load_gather(hist, [d])
                addr = base + cnt - 1
                plsc.store_scatter(dstk, [addr], vk, mask=valid)
                plsc.store_scatter(dsti, [addr], vi, mask=valid)
                plsc.store_scatter(hist, [d], base + cnt, mask=last)
                return c

            lax.fori_loop(0, nv, pp, jnp.int32(0))

        radix_pass(candk, candi, bufk, bufi, 0)
        radix_pass(bufk, bufi, candk, candi, 8)
        radix_pass(candk, candi, bufk, bufi, 16)
        radix_pass(bufk, bufi, candk, candi, 24)

        base = n - _K
        for j in range(_K // _L):
            outv[pl.ds(j * _L, _L)] = candi[pl.ds((base + j * _L)
                                                  & (_CAP - 1), _L)]
            onesv[pl.ds(j * _L, _L)] = jnp.full((_L,), 1.0, jnp.float32)
        pltpu.sync_copy(outv, idx_hbm.at[row])
        pltpu.sync_copy(onesv, ones_hbm.at[row])


def _sc_topk(keys):
    mesh = plsc.VectorSubcoreMesh(core_axis_name="c", subcore_axis_name="s",
                                  num_cores=1)
    return pl.kernel(
        _sc_body,
        out_type=(
            jax.ShapeDtypeStruct((_B, _K), jnp.int32),
            jax.ShapeDtypeStruct((_B, _K), jnp.float32),
        ),
        mesh=mesh,
        compiler_params=pltpu.CompilerParams(needs_layout_passes=False),
        scratch_types=[
            pltpu.VMEM((_N,), jnp.float32),
            pltpu.VMEM((4 * 256,), jnp.int32),
            pltpu.VMEM((_CAP,), jnp.int32),
            pltpu.VMEM((_CAP,), jnp.int32),
            pltpu.VMEM((_CAP,), jnp.int32),
            pltpu.VMEM((_CAP,), jnp.int32),
            pltpu.VMEM((_K,), jnp.int32),
            pltpu.VMEM((_K,), jnp.float32),
        ],
    )(keys)


def kernel(x, routing_token, num_tokens):
    del num_tokens
    scores = jnp.einsum('bnd,d->bn', x, routing_token)
    idx, ones = _sc_topk(scores)
    return ones, idx
